# baseline (speedup 1.0000x reference)
import functools
import jax
import jax.numpy as jnp
from jax import lax
from jax.experimental import pallas as pl
from jax.experimental.pallas import tpu as pltpu
from jax.experimental.pallas import tpu_sc as plsc

_N = 10000
_E = 160000
_DF = 128
_DE = 16
_HID = 16

_NC = 2
_NS = 16
_NW = _NC * _NS
_CHUNK = 80
_NBUF = 5
_BE = 1600
_RPAD = 20480


def _sc_mesh():
    return plsc.VectorSubcoreMesh(core_axis_name="c", subcore_axis_name="s",
                                  num_cores=_NC, num_subcores=_NS)


def _sc_gather_call(table, idx3, d):
    n_chunks = idx3.shape[1]
    per_w = n_chunks * _CHUNK
    e_tot = _NW * per_w
    n_groups = n_chunks // _NBUF

    def body(table_hbm, idx_hbm, out_hbm, idx_v, *rest):
        bufs = rest[:_NBUF]
        sems = rest[_NBUF:2 * _NBUF]
        wid = lax.axis_index("s") * _NC + lax.axis_index("c")
        base = wid * per_w
        pltpu.sync_copy(idx_hbm.at[wid], idx_v)

        def start(j, b):
            pltpu.async_copy(table_hbm.at[idx_v.at[j]], bufs[b], sems[b])

        def finish(j, b):
            pltpu.make_async_copy(table_hbm.at[idx_v.at[0]],
                                  bufs[b], sems[b]).wait()
            off = pl.multiple_of(base + j * _CHUNK, _CHUNK)
            pltpu.sync_copy(bufs[b], out_hbm.at[pl.ds(off, _CHUNK)])

        for b in range(_NBUF):
            start(b, b)

        def group(g, _):
            for b in range(_NBUF):
                j = g * _NBUF + b
                finish(j, b)
                start(j + _NBUF, b)
            return 0

        lax.fori_loop(0, n_groups - 1, group, 0)
        for b in range(_NBUF):
            finish((n_groups - 1) * _NBUF + b, b)

    return pl.kernel(
        body,
        out_type=jax.ShapeDtypeStruct((e_tot, d), jnp.float32),
        mesh=_sc_mesh(),
        compiler_params=pltpu.CompilerParams(
            use_tc_tiling_on_sc=(d % 128 == 0)),
        scratch_types=(
            [pltpu.VMEM((n_chunks, _CHUNK), jnp.int32)]
            + [pltpu.VMEM((_CHUNK, d), jnp.float32) for _ in range(_NBUF)]
            + [pltpu.SemaphoreType.DMA for _ in range(_NBUF)]
        ),
    )(table, idx3)


def _sc_scatter_call(msg, dst3, zeros, o):
    n_chunks = dst3.shape[1]
    per_w = n_chunks * _CHUNK
    n_groups = n_chunks // _NBUF
    rows = zeros.shape[0]
    rpt = rows // _NS

    def body(msg_hbm, dst_hbm, zero_hbm, out_hbm, idx_v, acc, *rest):
        bufs = rest[:_NBUF]
        lsems = rest[_NBUF:2 * _NBUF]
        ssems = rest[2 * _NBUF:3 * _NBUF]
        cid = lax.axis_index("c")
        sid = lax.axis_index("s")
        wid = sid * _NC + cid
        base = wid * per_w
        pltpu.sync_copy(dst_hbm.at[wid], idx_v)
        r0 = pl.multiple_of(sid * rpt, rpt)
        pltpu.sync_copy(zero_hbm.at[pl.ds(r0, rpt)], acc.at[pl.ds(r0, rpt)])
        plsc.subcore_barrier()

        def start_load(j, b):
            off = pl.multiple_of(base + j * _CHUNK, _CHUNK)
            pltpu.async_copy(msg_hbm.at[pl.ds(off, _CHUNK)],
                             bufs[b], lsems[b])

        def wait_load(b):
            pltpu.make_async_copy(msg_hbm.at[pl.ds(0, _CHUNK)],
                                  bufs[b], lsems[b]).wait()

        def scat(j, b):
            pltpu.async_copy(bufs[b], acc.at[idx_v.at[j]], ssems[b],
                             add=True)
            pltpu.make_async_copy(bufs[b], acc.at[idx_v.at[0]],
                                  ssems[b]).wait()

        for b in range(_NBUF):
            start_load(b, b)

        def group(g, _):
            for b in range(_NBUF):
                j = g * _NBUF + b
                wait_load(b)
                scat(j, b)
                start_load(j + _NBUF, b)
            return 0

        lax.fori_loop(0, n_groups - 1, group, 0)
        for b in range(_NBUF):
            wait_load(b)
            scat((n_groups - 1) * _NBUF + b, b)
        plsc.subcore_barrier()
        pltpu.sync_copy(acc.at[pl.ds(r0, rpt)],
                        out_hbm.at[cid].at[pl.ds(r0, rpt)])

    return pl.kernel(
        body,
        out_type=jax.ShapeDtypeStruct((_NC, rows, o), jnp.float32),
        mesh=_sc_mesh(),
        compiler_params=pltpu.CompilerParams(use_tc_tiling_on_sc=False),
        scratch_types=(
            [pltpu.VMEM((n_chunks, _CHUNK), jnp.int32),
             pltpu.VMEM_SHARED((rows, o), jnp.float32)]
            + [pltpu.VMEM((_CHUNK, o), jnp.float32) for _ in range(_NBUF)]
            + [pltpu.SemaphoreType.DMA for _ in range(2 * _NBUF)]
        ),
    )(msg, dst3, zeros)


def _build_w2aug(p, in_dim, out_dim):
    w2r = p['enW2'].reshape(_HID, in_dim, out_dim)
    b2r = p['enb2'].reshape(in_dim, out_dim)
    full = jnp.concatenate([jnp.transpose(w2r, (1, 2, 0)), b2r[:, :, None]],
                           axis=2)
    return full.reshape(in_dim, out_dim * (_HID + 1))


def _msg_body(xj_ref, ea_ref, w1_ref, b1_ref, w2_ref, r16_ref, m16_ref,
              s_ref, msg_ref):
    h = jnp.maximum(jnp.dot(ea_ref[0], w1_ref[0],
                            preferred_element_type=jnp.float32) + b1_ref[0], 0.0)
    t2 = jnp.dot(xj_ref[...], w2_ref[0], preferred_element_type=jnp.float32)
    ht = jnp.dot(h, r16_ref[...], preferred_element_type=jnp.float32) \
        + m16_ref[...]
    msg_ref[...] = jnp.dot(ht * t2, s_ref[...],
                           preferred_element_type=jnp.float32)


def _msg_call(xjp, eap, w1_s, b1_s, w2_s, in_dim, out_dim, pack):
    be = _BE
    rows = be // pack
    nblk = (2 * _E) // be
    half = nblk // 2
    kk = _HID + 1
    eyep = jnp.eye(pack, dtype=jnp.float32)
    eye = jnp.eye(kk, dtype=jnp.float32)
    r16 = jnp.kron(eyep, jnp.tile(eye[:_HID], (1, out_dim)))
    m16 = jnp.tile(jnp.tile(eye[_HID], (out_dim,)),
                   (pack,)).reshape(1, pack * out_dim * kk)
    s = jnp.kron(eyep, jnp.repeat(jnp.eye(out_dim, dtype=jnp.float32),
                                  kk, axis=0))
    w1k = jnp.stack([jnp.kron(eyep, w1_s[0]), jnp.kron(eyep, w1_s[1])])
    b1k = jnp.tile(b1_s, (1, 1, pack))
    w2k = jnp.stack([jnp.kron(eyep, w2_s[0]), jnp.kron(eyep, w2_s[1])])
    pin, pde, pko = pack * in_dim, pack * _DE, pack * kk * out_dim
    return pl.pallas_call(
        _msg_body,
        grid=(nblk,),
        in_specs=[
            pl.BlockSpec((rows, pin), lambda i: (i, 0)),
            pl.BlockSpec((1, rows, pde), lambda i: (i // half, i % half, 0)),
            pl.BlockSpec((1, pde, pde), lambda i: (i // half, 0, 0)),
            pl.BlockSpec((1, 1, pde), lambda i: (i // half, 0, 0)),
            pl.BlockSpec((1, pin, pko), lambda i: (i // half, 0, 0)),
            pl.BlockSpec((pde, pko), lambda i: (0, 0)),
            pl.BlockSpec((1, pko), lambda i: (0, 0)),
            pl.BlockSpec((pko, pack * out_dim), lambda i: (0, 0)),
        ],
        out_specs=pl.BlockSpec((rows, pack * out_dim), lambda i: (i, 0)),
        out_shape=jax.ShapeDtypeStruct(((2 * _E) // pack, pack * out_dim),
                                       jnp.float32),
    )(xjp, eap, w1k, b1k, w2k, r16, m16, s)


def _upd_body(part_ref, x_ref, root_ref, bias_ref, out_ref):
    out_ref[...] = jnp.maximum(
        part_ref[0] + part_ref[1]
        + jnp.dot(x_ref[...], root_ref[0], preferred_element_type=jnp.float32)
        + bias_ref[0], 0.0)


def _upd_call(part, x, root_s, bias_s, in_dim, out_dim):
    return pl.pallas_call(
        _upd_body,
        grid=(2,),
        in_specs=[
            pl.BlockSpec((_NC, _N, out_dim), lambda i: (0, i, 0)),
            pl.BlockSpec((_N, in_dim), lambda i: (i, 0)),
            pl.BlockSpec((1, in_dim, out_dim), lambda i: (i, 0, 0)),
            pl.BlockSpec((1, 1, out_dim), lambda i: (i, 0, 0)),
        ],
        out_specs=pl.BlockSpec((_N, out_dim), lambda i: (i, 0)),
        out_shape=jax.ShapeDtypeStruct((2 * _N, out_dim), jnp.float32),
    )(part, x, root_s, bias_s)


def _head_body(hp_ref, hd_ref, pw1_ref, pb1_ref, pw2_ref, pb2_ref,
               l1a_ref, l1c_ref, l1b_ref, l2w_ref, l2b_ref, out_ref):
    def pool(x):
        g = jnp.dot(
            jnp.maximum(jnp.dot(x, pw1_ref[...],
                                preferred_element_type=jnp.float32)
                        + pb1_ref[...], 0.0),
            pw2_ref[...], preferred_element_type=jnp.float32) + pb2_ref[...]
        m = jnp.max(g)
        e = jnp.exp(g - m)
        a = e / jnp.sum(e)
        return jnp.sum(a * x, axis=0, keepdims=True)

    zp = pool(hp_ref[...])
    zd = pool(hd_ref[...])
    r = jnp.maximum(
        jnp.dot(zp, l1a_ref[...], preferred_element_type=jnp.float32)
        + jnp.dot(zd, l1c_ref[...], preferred_element_type=jnp.float32)
        + l1b_ref[...], 0.0)
    out_ref[...] = jnp.dot(r, l2w_ref[...],
                           preferred_element_type=jnp.float32) + l2b_ref[...]


def _head_call(hp, hd, pool_p, l1w, l1b, l2w, l2b):
    return pl.pallas_call(
        _head_body,
        out_shape=jax.ShapeDtypeStruct((1, 1), jnp.float32),
    )(hp, hd,
      pool_p['W1'], pool_p['b1'].reshape(1, _HID),
      pool_p['W2'], pool_p['b2'].reshape(1, 1),
      l1w[:_HID], l1w[_HID:], l1b.reshape(1, 8),
      l2w, l2b.reshape(1, 1))


def kernel(x_p, x_d, edge_attr_p, edge_attr_d, edge_index_p, edge_index_d,
           params):
    x_cat = jnp.concatenate([x_p, x_d], axis=0)
    src = jnp.concatenate([edge_index_p[0], edge_index_d[0] + _N])
    dst = jnp.concatenate([edge_index_p[1], edge_index_d[1] + _N])
    n_chunks = (2 * _E) // (_NW * _CHUNK)
    src3 = src.reshape(_NW, n_chunks, _CHUNK)
    dst3 = dst.reshape(_NW, n_chunks, _CHUNK)

    h = x_cat
    for kp, kd, in_dim, out_dim, pack in (('p1', 'd1', _DF, 32, 4),
                                          ('p2', 'd2', 32, 16, 8)):
        pp, dd = params[kp], params[kd]
        w1_s = jnp.stack([pp['enW1'], dd['enW1']])
        b1_s = jnp.stack([pp['enb1'], dd['enb1']]).reshape(2, 1, _HID)
        w2_s = jnp.stack([_build_w2aug(pp, in_dim, out_dim),
                          _build_w2aug(dd, in_dim, out_dim)])
        root_s = jnp.stack([pp['root'], dd['root']])
        bias_s = jnp.stack([pp['bias'], dd['bias']]).reshape(2, 1, out_dim)
        eap = jnp.stack([edge_attr_p.reshape(_E // pack, pack * _DE),
                         edge_attr_d.reshape(_E // pack, pack * _DE)])

        xj = _sc_gather_call(h, src3, in_dim)
        xjp = xj.reshape((2 * _E) // pack, pack * in_dim)
        msgp = _msg_call(xjp, eap, w1_s, b1_s, w2_s, in_dim, out_dim, pack)
        msg = msgp.reshape(2 * _E, out_dim)
        zeros = jnp.zeros((_RPAD, out_dim), jnp.float32)
        part = _sc_scatter_call(msg, dst3, zeros, out_dim)
        h = _upd_call(part, h, root_s, bias_s, in_dim, out_dim)

    return _head_call(h[:_N], h[_N:], params['pool'],
                      params['lin1_W'], params['lin1_b'],
                      params['lin2_W'], params['lin2_b'])


# speedup vs baseline: 1.2742x; 1.2742x over previous
import functools
import numpy as np
import jax
import jax.numpy as jnp
from jax import lax
from jax.experimental import pallas as pl
from jax.experimental.pallas import tpu as pltpu
from jax.experimental.pallas import tpu_sc as plsc

_N = 10000
_E = 160000
_DF = 128
_DE = 16
_HID = 16

_NC = 2
_NS = 16
_NW = _NC * _NS
_CHUNK = 80
_NBUF = 5
_BE = 1600
_RPAD = 20480


def _sc_mesh():
    return plsc.VectorSubcoreMesh(core_axis_name="c", subcore_axis_name="s",
                                  num_cores=_NC, num_subcores=_NS)


def _sc_gather_call(table, idx3, d):
    n_chunks = idx3.shape[1]
    per_w = n_chunks * _CHUNK
    e_tot = _NW * per_w
    n_groups = n_chunks // _NBUF

    def body(table_hbm, idx_hbm, out_hbm, idx_v, *rest):
        bufs = rest[:_NBUF]
        sems = rest[_NBUF:2 * _NBUF]
        wid = lax.axis_index("s") * _NC + lax.axis_index("c")
        base = wid * per_w
        pltpu.sync_copy(idx_hbm.at[wid], idx_v)

        def start(j, b):
            pltpu.async_copy(table_hbm.at[idx_v.at[j]], bufs[b], sems[b])

        def finish(j, b):
            pltpu.make_async_copy(table_hbm.at[idx_v.at[0]],
                                  bufs[b], sems[b]).wait()
            off = pl.multiple_of(base + j * _CHUNK, _CHUNK)
            pltpu.sync_copy(bufs[b], out_hbm.at[pl.ds(off, _CHUNK)])

        for b in range(_NBUF):
            start(b, b)

        def group(g, _):
            for b in range(_NBUF):
                j = g * _NBUF + b
                finish(j, b)
                start(j + _NBUF, b)
            return 0

        lax.fori_loop(0, n_groups - 1, group, 0)
        for b in range(_NBUF):
            finish((n_groups - 1) * _NBUF + b, b)

    return pl.kernel(
        body,
        out_type=jax.ShapeDtypeStruct((e_tot, d), jnp.float32),
        mesh=_sc_mesh(),
        compiler_params=pltpu.CompilerParams(
            use_tc_tiling_on_sc=(d % 128 == 0)),
        scratch_types=(
            [pltpu.VMEM((n_chunks, _CHUNK), jnp.int32)]
            + [pltpu.VMEM((_CHUNK, d), jnp.float32) for _ in range(_NBUF)]
            + [pltpu.SemaphoreType.DMA for _ in range(_NBUF)]
        ),
    )(table, idx3)


def _sc_scatter_call(msg, dst3, zeros, o):
    n_chunks = dst3.shape[1]
    per_w = n_chunks * _CHUNK
    n_groups = n_chunks // _NBUF
    rows = zeros.shape[0]
    rpt = rows // _NS

    def body(msg_hbm, dst_hbm, zero_hbm, out_hbm, idx_v, acc, *rest):
        bufs = rest[:_NBUF]
        lsems = rest[_NBUF:2 * _NBUF]
        ssems = rest[2 * _NBUF:3 * _NBUF]
        cid = lax.axis_index("c")
        sid = lax.axis_index("s")
        wid = sid * _NC + cid
        base = wid * per_w
        pltpu.sync_copy(dst_hbm.at[wid], idx_v)
        r0 = pl.multiple_of(sid * rpt, rpt)
        pltpu.sync_copy(zero_hbm.at[pl.ds(r0, rpt)], acc.at[pl.ds(r0, rpt)])
        plsc.subcore_barrier()

        def start_load(j, b):
            off = pl.multiple_of(base + j * _CHUNK, _CHUNK)
            pltpu.async_copy(msg_hbm.at[pl.ds(off, _CHUNK)],
                             bufs[b], lsems[b])

        def wait_load(b):
            pltpu.make_async_copy(msg_hbm.at[pl.ds(0, _CHUNK)],
                                  bufs[b], lsems[b]).wait()

        def scat(j, b):
            pltpu.async_copy(bufs[b], acc.at[idx_v.at[j]], ssems[b],
                             add=True)
            pltpu.make_async_copy(bufs[b], acc.at[idx_v.at[0]],
                                  ssems[b]).wait()

        for b in range(_NBUF):
            start_load(b, b)

        def group(g, _):
            for b in range(_NBUF):
                j = g * _NBUF + b
                wait_load(b)
                scat(j, b)
                start_load(j + _NBUF, b)
            return 0

        lax.fori_loop(0, n_groups - 1, group, 0)
        for b in range(_NBUF):
            wait_load(b)
            scat((n_groups - 1) * _NBUF + b, b)
        plsc.subcore_barrier()
        pltpu.sync_copy(acc.at[pl.ds(r0, rpt)],
                        out_hbm.at[cid].at[pl.ds(r0, rpt)])

    return pl.kernel(
        body,
        out_type=jax.ShapeDtypeStruct((_NC, rows, o), jnp.float32),
        mesh=_sc_mesh(),
        compiler_params=pltpu.CompilerParams(use_tc_tiling_on_sc=False),
        scratch_types=(
            [pltpu.VMEM((n_chunks, _CHUNK), jnp.int32),
             pltpu.VMEM_SHARED((rows, o), jnp.float32)]
            + [pltpu.VMEM((_CHUNK, o), jnp.float32) for _ in range(_NBUF)]
            + [pltpu.SemaphoreType.DMA for _ in range(2 * _NBUF)]
        ),
    )(msg, dst3, zeros)


def _build_w2aug(p, in_dim, out_dim):
    w2r = p['enW2'].reshape(_HID, in_dim, out_dim)
    b2r = p['enb2'].reshape(in_dim, out_dim)
    full = jnp.concatenate([jnp.transpose(w2r, (1, 2, 0)), b2r[:, :, None]],
                           axis=2)
    return full.reshape(in_dim, out_dim * (_HID + 1))


def _msg_body(xj_ref, ea_ref, w1_ref, b1_ref, w2_ref, r16_ref, m16_ref,
              s_ref, msg_ref):
    h = jnp.maximum(jnp.dot(ea_ref[0], w1_ref[0],
                            preferred_element_type=jnp.float32) + b1_ref[0], 0.0)
    t2 = jnp.dot(xj_ref[...], w2_ref[0], preferred_element_type=jnp.float32)
    ht = jnp.dot(h, r16_ref[...], preferred_element_type=jnp.float32) \
        + m16_ref[...]
    msg_ref[...] = jnp.dot(ht * t2, s_ref[...],
                           preferred_element_type=jnp.float32)


def _msg1_body(half, qs, rows, w544, xj_ref, eap_ref, ead_ref, w1_ref,
               b1_ref, w2_ref, r16_ref, m16_ref, sq_ref, msg_ref):
    i = pl.program_id(0)
    ea = jnp.where(i < half, eap_ref[...], ead_ref[...])
    h = jnp.maximum(jnp.dot(ea, w1_ref[0],
                            preferred_element_type=jnp.float32) + b1_ref[0], 0.0)
    t2 = jnp.dot(xj_ref[...], w2_ref[0], preferred_element_type=jnp.float32)
    ht = jnp.dot(h, r16_ref[...], preferred_element_type=jnp.float32) \
        + m16_ref[...]
    p = ht * t2
    acc = jnp.zeros((rows, 128), jnp.float32)
    for q in range(qs):
        acc = acc + jnp.dot(p[q * rows:(q + 1) * rows, :], sq_ref[q],
                            preferred_element_type=jnp.float32)
    msg_ref[...] = acc


def _msg1_call(xj, ea_p, ea_d, w1_s, b1_s, w2_s, in_dim, out_dim):
    be = _BE
    qs = 128 // out_dim
    rows = be // qs
    nblk = (2 * _E) // be
    half = nblk // 2
    kk = _HID + 1
    eye = jnp.eye(kk, dtype=jnp.float32)
    r16 = jnp.tile(eye[:_HID], (1, out_dim))
    m16 = jnp.tile(eye[_HID], (out_dim,)).reshape(1, out_dim * kk)
    s_one = jnp.repeat(jnp.eye(out_dim, dtype=jnp.float32), kk, axis=0)
    sq = jnp.stack([
        jnp.pad(s_one, ((0, 0), (q * out_dim, (qs - 1 - q) * out_dim)))
        for q in range(qs)])
    ko = kk * out_dim
    return pl.pallas_call(
        functools.partial(_msg1_body, half, qs, rows, ko),
        grid=(nblk,),
        in_specs=[
            pl.BlockSpec((be, in_dim), lambda i: (i, 0)),
            pl.BlockSpec((be, _DE), lambda i: (jnp.minimum(i, half - 1), 0)),
            pl.BlockSpec((be, _DE),
                         lambda i: (jnp.maximum(i - half, 0), 0)),
            pl.BlockSpec((1, _DE, _HID), lambda i: (i // half, 0, 0)),
            pl.BlockSpec((1, 1, _HID), lambda i: (i // half, 0, 0)),
            pl.BlockSpec((1, in_dim, ko), lambda i: (i // half, 0, 0)),
            pl.BlockSpec((_DE, ko), lambda i: (0, 0)),
            pl.BlockSpec((1, ko), lambda i: (0, 0)),
            pl.BlockSpec((qs, ko, 128), lambda i: (0, 0, 0)),
        ],
        out_specs=pl.BlockSpec((rows, 128), lambda i: (i, 0)),
        out_shape=jax.ShapeDtypeStruct(((2 * _E) // qs, 128), jnp.float32),
    )(xj, ea_p, ea_d, w1_s, b1_s, w2_s, r16, m16, sq)


def _msg1_perm(be, qs):
    ep = np.arange(2 * _E)
    rr = ep // qs
    q = ep % qs
    i = rr // (be // qs)
    r = rr % (be // qs)
    return jnp.asarray(i * be + q * (be // qs) + r, dtype=jnp.int32)


def _msg_call(xjp, eap, w1_s, b1_s, w2_s, in_dim, out_dim, pack):
    be = _BE
    rows = be // pack
    nblk = (2 * _E) // be
    half = nblk // 2
    kk = _HID + 1
    eyep = jnp.eye(pack, dtype=jnp.float32)
    eye = jnp.eye(kk, dtype=jnp.float32)
    r16 = jnp.kron(eyep, jnp.tile(eye[:_HID], (1, out_dim)))
    m16 = jnp.tile(jnp.tile(eye[_HID], (out_dim,)),
                   (pack,)).reshape(1, pack * out_dim * kk)
    s = jnp.kron(eyep, jnp.repeat(jnp.eye(out_dim, dtype=jnp.float32),
                                  kk, axis=0))
    w1k = jnp.stack([jnp.kron(eyep, w1_s[0]), jnp.kron(eyep, w1_s[1])])
    b1k = jnp.tile(b1_s, (1, 1, pack))
    w2k = jnp.stack([jnp.kron(eyep, w2_s[0]), jnp.kron(eyep, w2_s[1])])
    pin, pde, pko = pack * in_dim, pack * _DE, pack * kk * out_dim
    return pl.pallas_call(
        _msg_body,
        grid=(nblk,),
        in_specs=[
            pl.BlockSpec((rows, pin), lambda i: (i, 0)),
            pl.BlockSpec((1, rows, pde), lambda i: (i // half, i % half, 0)),
            pl.BlockSpec((1, pde, pde), lambda i: (i // half, 0, 0)),
            pl.BlockSpec((1, 1, pde), lambda i: (i // half, 0, 0)),
            pl.BlockSpec((1, pin, pko), lambda i: (i // half, 0, 0)),
            pl.BlockSpec((pde, pko), lambda i: (0, 0)),
            pl.BlockSpec((1, pko), lambda i: (0, 0)),
            pl.BlockSpec((pko, pack * out_dim), lambda i: (0, 0)),
        ],
        out_specs=pl.BlockSpec((rows, pack * out_dim), lambda i: (i, 0)),
        out_shape=jax.ShapeDtypeStruct(((2 * _E) // pack, pack * out_dim),
                                       jnp.float32),
    )(xjp, eap, w1k, b1k, w2k, r16, m16, s)


def _upd_body(part_ref, x_ref, root_ref, bias_ref, out_ref):
    out_ref[...] = jnp.maximum(
        part_ref[0] + part_ref[1]
        + jnp.dot(x_ref[...], root_ref[0], preferred_element_type=jnp.float32)
        + bias_ref[0], 0.0)


def _upd_call(part, x, root_s, bias_s, in_dim, out_dim):
    return pl.pallas_call(
        _upd_body,
        grid=(2,),
        in_specs=[
            pl.BlockSpec((_NC, _N, out_dim), lambda i: (0, i, 0)),
            pl.BlockSpec((_N, in_dim), lambda i: (i, 0)),
            pl.BlockSpec((1, in_dim, out_dim), lambda i: (i, 0, 0)),
            pl.BlockSpec((1, 1, out_dim), lambda i: (i, 0, 0)),
        ],
        out_specs=pl.BlockSpec((_N, out_dim), lambda i: (i, 0)),
        out_shape=jax.ShapeDtypeStruct((2 * _N, out_dim), jnp.float32),
    )(part, x, root_s, bias_s)


def _head_body(hp_ref, hd_ref, pw1_ref, pb1_ref, pw2_ref, pb2_ref,
               l1a_ref, l1c_ref, l1b_ref, l2w_ref, l2b_ref, out_ref):
    def pool(x):
        g = jnp.dot(
            jnp.maximum(jnp.dot(x, pw1_ref[...],
                                preferred_element_type=jnp.float32)
                        + pb1_ref[...], 0.0),
            pw2_ref[...], preferred_element_type=jnp.float32) + pb2_ref[...]
        m = jnp.max(g)
        e = jnp.exp(g - m)
        a = e / jnp.sum(e)
        return jnp.sum(a * x, axis=0, keepdims=True)

    zp = pool(hp_ref[...])
    zd = pool(hd_ref[...])
    r = jnp.maximum(
        jnp.dot(zp, l1a_ref[...], preferred_element_type=jnp.float32)
        + jnp.dot(zd, l1c_ref[...], preferred_element_type=jnp.float32)
        + l1b_ref[...], 0.0)
    out_ref[...] = jnp.dot(r, l2w_ref[...],
                           preferred_element_type=jnp.float32) + l2b_ref[...]


def _head_call(hp, hd, pool_p, l1w, l1b, l2w, l2b):
    return pl.pallas_call(
        _head_body,
        out_shape=jax.ShapeDtypeStruct((1, 1), jnp.float32),
    )(hp, hd,
      pool_p['W1'], pool_p['b1'].reshape(1, _HID),
      pool_p['W2'], pool_p['b2'].reshape(1, 1),
      l1w[:_HID], l1w[_HID:], l1b.reshape(1, 8),
      l2w, l2b.reshape(1, 1))


def kernel(x_p, x_d, edge_attr_p, edge_attr_d, edge_index_p, edge_index_d,
           params):
    x_cat = jnp.concatenate([x_p, x_d], axis=0)
    src = jnp.concatenate([edge_index_p[0], edge_index_d[0] + _N])
    dst = jnp.concatenate([edge_index_p[1], edge_index_d[1] + _N])
    n_chunks = (2 * _E) // (_NW * _CHUNK)
    src3 = src.reshape(_NW, n_chunks, _CHUNK)
    dst3 = dst.reshape(_NW, n_chunks, _CHUNK)

    def layer_weights(kp, kd, in_dim, out_dim):
        pp, dd = params[kp], params[kd]
        return (jnp.stack([pp['enW1'], dd['enW1']]),
                jnp.stack([pp['enb1'], dd['enb1']]).reshape(2, 1, _HID),
                jnp.stack([_build_w2aug(pp, in_dim, out_dim),
                           _build_w2aug(dd, in_dim, out_dim)]),
                jnp.stack([pp['root'], dd['root']]),
                jnp.stack([pp['bias'], dd['bias']]).reshape(2, 1, out_dim))

    w1_s, b1_s, w2_s, root_s, bias_s = layer_weights('p1', 'd1', _DF, 32)
    xj = _sc_gather_call(x_cat, src3, _DF)
    msgp = _msg1_call(xj, edge_attr_p, edge_attr_d, w1_s, b1_s, w2_s,
                      _DF, 32)
    msg = msgp.reshape(2 * _E, 32)
    dst1 = dst[_msg1_perm(_BE, 4)].reshape(_NW, n_chunks, _CHUNK)
    part = _sc_scatter_call(msg, dst1, jnp.zeros((_RPAD, 32), jnp.float32),
                            32)
    h = _upd_call(part, x_cat, root_s, bias_s, _DF, 32)

    pack = 8
    w1_s, b1_s, w2_s, root_s, bias_s = layer_weights('p2', 'd2', 32, 16)
    eap = jnp.stack([edge_attr_p.reshape(_E // pack, pack * _DE),
                     edge_attr_d.reshape(_E // pack, pack * _DE)])
    xj2 = _sc_gather_call(h, src3, 32)
    xjp = xj2.reshape((2 * _E) // pack, pack * 32)
    msgp = _msg_call(xjp, eap, w1_s, b1_s, w2_s, 32, 16, pack)
    msg = msgp.reshape(2 * _E, 16)
    part = _sc_scatter_call(msg, dst3, jnp.zeros((_RPAD, 16), jnp.float32),
                            16)
    h = _upd_call(part, h, root_s, bias_s, 32, 16)

    return _head_call(h[:_N], h[_N:], params['pool'],
                      params['lin1_W'], params['lin1_b'],
                      params['lin2_W'], params['lin2_b'])
